# baseline (speedup 1.0000x reference)
import functools

import jax
import jax.numpy as jnp
from jax import lax
from jax.experimental import pallas as pl
from jax.experimental.pallas import tpu as pltpu
from jax.experimental.pallas import tpu_sc as plsc

B = 16384
D = 64
NROW = 1000000
RC = 16384
NCHUNK = (NROW + RC - 1) // RC
NPACK = NCHUNK * RC

_info = plsc.get_sparse_core_info()
_NC, _NS = _info.num_cores, _info.num_subcores
_NW = _NC * _NS
_BPW = B // _NW


def _pack_body(xa_ref, xb_ref, eye_ref, o_ref):
    va = xa_ref[...].reshape(D, RC)
    vb = xb_ref[...].reshape(D, RC)
    del eye_ref
    v = jnp.concatenate([va, vb], axis=0)
    t = v.T
    b16 = t.astype(jnp.bfloat16)
    o_ref[...] = pltpu.bitcast(b16, jnp.float32)


def _tc_pack(view_a, view_b, eye):
    return pl.pallas_call(
        _pack_body,
        grid=(NCHUNK,),
        in_specs=[pl.BlockSpec((8, 8, RC), lambda i: (0, 0, i)),
                  pl.BlockSpec((8, 8, RC), lambda i: (0, 0, i)),
                  pl.BlockSpec((128, 128), lambda i: (0, 0))],
        out_specs=pl.BlockSpec((RC // 2, 128), lambda i: (i, 0)),
        out_shape=jax.ShapeDtypeStruct((NPACK // 2, 128), jnp.float32),
        compiler_params=pltpu.CompilerParams(
            dimension_semantics=("arbitrary",)),
    )(view_a, view_b, eye)


def _sc_gather(idx, tbl):
    mesh = plsc.VectorSubcoreMesh(core_axis_name="c", subcore_axis_name="s")
    f32 = jnp.float32

    @functools.partial(
        pl.kernel,
        mesh=mesh,
        compiler_params=pltpu.CompilerParams(
            use_tc_tiling_on_sc=True, needs_layout_passes=False),
        out_type=jax.ShapeDtypeStruct((B, 128), f32),
        scratch_types=[
            pltpu.VMEM((_BPW,), jnp.int32),
            pltpu.VMEM((4, 128), jnp.int32),
            pltpu.VMEM((_BPW, 128), f32),
            pltpu.SemaphoreType.DMA,
        ],
    )
    def k(idx_hbm, tbl_hbm, out, idx_v, pidx_v, stage, s0):
        wid = lax.axis_index("s") * _NC + lax.axis_index("c")
        base = wid * _BPW
        pltpu.sync_copy(idx_hbm.at[pl.ds(base, _BPW)], idx_v)
        for c in range(_BPW // 16):
            pidx_v[c // 8, pl.ds((c % 8) * 16, 16)] = (
                lax.shift_right_logical(idx_v[pl.ds(c * 16, 16)], 1))
        cps = [pltpu.async_copy(tbl_hbm.at[pidx_v.at[g]],
                                stage.at[pl.ds(g * 128, 128)], s0)
               for g in range(4)]
        for cp in cps:
            cp.wait()
        pltpu.sync_copy(stage, out.at[pl.ds(base, _BPW)])

    return k(idx, tbl)


_BS = 2048
_G = B // _BS


def _unpack_rows(words, parity):
    wi = lax.bitcast_convert_type(words, jnp.int32)
    lo = lax.bitcast_convert_type(lax.shift_left(wi, 16), jnp.float32)
    hi = lax.bitcast_convert_type(
        jnp.bitwise_and(wi, jnp.int32(-65536)), jnp.float32)
    return jnp.where(parity == 1, hi, lo)


def _tc_body(gu, gi, pu, pi, w1a, w1b, b1, w2, b2, woa, wob, bo, out):
    u = _unpack_rows(gu[...], pu[...])
    i = _unpack_rows(gi[...], pi[...])
    h1 = jnp.maximum(
        jnp.dot(u[:, :D], w1a[...], preferred_element_type=jnp.float32)
        + jnp.dot(i[:, :D], w1b[...], preferred_element_type=jnp.float32)
        + b1[...], 0.0)
    h2 = jnp.maximum(
        jnp.dot(h1, w2[...], preferred_element_type=jnp.float32) + b2[...],
        0.0)
    mf = u[:, D:] * i[:, D:]
    logit = (jnp.dot(h2, woa[...], preferred_element_type=jnp.float32)
             + jnp.dot(mf, wob[...], preferred_element_type=jnp.float32)
             + bo[...])
    out[...] = jax.nn.sigmoid(logit)


def _tc_dense(gU, gI, parU, parI, W1, b1, W2, b2, Wo, bo):
    w1a, w1b = W1[:D], W1[D:]
    woa, wob = Wo[:32], Wo[32:]
    b1r = b1.reshape(1, -1)
    b2r = b2.reshape(1, -1)
    bor = bo.reshape(1, 1)
    row_spec = pl.BlockSpec((_BS, 128), lambda i: (i, 0))
    par_spec = pl.BlockSpec((_BS, 1), lambda i: (i, 0))
    full = lambda a: pl.BlockSpec(a.shape, lambda i: (0,) * a.ndim)
    out = pl.pallas_call(
        _tc_body,
        grid=(_G,),
        in_specs=[row_spec, row_spec, par_spec, par_spec,
                  full(w1a), full(w1b), full(b1r), full(W2), full(b2r),
                  full(woa), full(wob), full(bor)],
        out_specs=pl.BlockSpec((_BS, 1), lambda i: (i, 0)),
        out_shape=jax.ShapeDtypeStruct((B, 1), jnp.float32),
        compiler_params=pltpu.CompilerParams(
            dimension_semantics=("arbitrary",)),
    )(gU, gI, parU, parI, w1a, w1b, b1r, W2, b2r, woa, wob, bor)
    return out.reshape(B)


def kernel(user_indices, item_indices, embed_user_mlp, embed_item_mlp,
           embed_user_mf, embed_item_mf, W1, b1, W2, b2, Wo, bo):
    vUm = embed_user_mlp.T.reshape(8, 8, NROW)
    vUf = embed_user_mf.T.reshape(8, 8, NROW)
    vIm = embed_item_mlp.T.reshape(8, 8, NROW)
    vIf = embed_item_mf.T.reshape(8, 8, NROW)
    eye = jnp.eye(128, dtype=jnp.float32)
    pU = _tc_pack(vUm, vUf, eye)
    pI = _tc_pack(vIm, vIf, eye)
    gU = _sc_gather(user_indices, pU)
    gI = _sc_gather(item_indices, pI)
    parU = jnp.bitwise_and(user_indices, 1).reshape(B, 1)
    parI = jnp.bitwise_and(item_indices, 1).reshape(B, 1)
    return _tc_dense(gU, gI, parU, parI, W1, b1, W2, b2, Wo, bo)


# speedup vs baseline: 1.0073x; 1.0073x over previous
import functools

import jax
import jax.numpy as jnp
from jax import lax
from jax.experimental import pallas as pl
from jax.experimental.pallas import tpu as pltpu
from jax.experimental.pallas import tpu_sc as plsc

B = 16384
D = 64
NROW = 1000000
RC = 32768
NCHUNK = (NROW + RC - 1) // RC
NPACK = NCHUNK * RC

_info = plsc.get_sparse_core_info()
_NC, _NS = _info.num_cores, _info.num_subcores
_NW = _NC * _NS
_BPW = B // _NW


def _pack_body(xa_ref, xb_ref, eye_ref, o_ref):
    va = xa_ref[...].reshape(D, RC)
    vb = xb_ref[...].reshape(D, RC)
    del eye_ref
    v = jnp.concatenate([va, vb], axis=0)
    t = v.T
    b16 = t.astype(jnp.bfloat16)
    o_ref[...] = pltpu.bitcast(b16, jnp.float32)


def _tc_pack(view_a, view_b, eye):
    return pl.pallas_call(
        _pack_body,
        grid=(NCHUNK,),
        in_specs=[pl.BlockSpec((8, 8, RC), lambda i: (0, 0, i)),
                  pl.BlockSpec((8, 8, RC), lambda i: (0, 0, i)),
                  pl.BlockSpec((128, 128), lambda i: (0, 0))],
        out_specs=pl.BlockSpec((RC // 2, 128), lambda i: (i, 0)),
        out_shape=jax.ShapeDtypeStruct((NPACK // 2, 128), jnp.float32),
        compiler_params=pltpu.CompilerParams(
            dimension_semantics=("arbitrary",)),
    )(view_a, view_b, eye)


def _sc_gather(idx, tbl):
    mesh = plsc.VectorSubcoreMesh(core_axis_name="c", subcore_axis_name="s")
    f32 = jnp.float32

    @functools.partial(
        pl.kernel,
        mesh=mesh,
        compiler_params=pltpu.CompilerParams(
            use_tc_tiling_on_sc=True, needs_layout_passes=False),
        out_type=jax.ShapeDtypeStruct((B, 128), f32),
        scratch_types=[
            pltpu.VMEM((_BPW,), jnp.int32),
            pltpu.VMEM((4, 128), jnp.int32),
            pltpu.VMEM((_BPW, 128), f32),
            pltpu.SemaphoreType.DMA,
        ],
    )
    def k(idx_hbm, tbl_hbm, out, idx_v, pidx_v, stage, s0):
        wid = lax.axis_index("s") * _NC + lax.axis_index("c")
        base = wid * _BPW
        pltpu.sync_copy(idx_hbm.at[pl.ds(base, _BPW)], idx_v)
        for c in range(_BPW // 16):
            pidx_v[c // 8, pl.ds((c % 8) * 16, 16)] = (
                lax.shift_right_logical(idx_v[pl.ds(c * 16, 16)], 1))
        cps = [pltpu.async_copy(tbl_hbm.at[pidx_v.at[g]],
                                stage.at[pl.ds(g * 128, 128)], s0)
               for g in range(4)]
        for cp in cps:
            cp.wait()
        pltpu.sync_copy(stage, out.at[pl.ds(base, _BPW)])

    return k(idx, tbl)


_BS = 2048
_G = B // _BS


def _unpack_rows(words, parity):
    wi = lax.bitcast_convert_type(words, jnp.int32)
    lo = lax.bitcast_convert_type(lax.shift_left(wi, 16), jnp.float32)
    hi = lax.bitcast_convert_type(
        jnp.bitwise_and(wi, jnp.int32(-65536)), jnp.float32)
    return jnp.where(parity == 1, hi, lo)


def _tc_body(gu, gi, pu, pi, w1a, w1b, b1, w2, b2, woa, wob, bo, out):
    u = _unpack_rows(gu[...], pu[...])
    i = _unpack_rows(gi[...], pi[...])
    h1 = jnp.maximum(
        jnp.dot(u[:, :D], w1a[...], preferred_element_type=jnp.float32)
        + jnp.dot(i[:, :D], w1b[...], preferred_element_type=jnp.float32)
        + b1[...], 0.0)
    h2 = jnp.maximum(
        jnp.dot(h1, w2[...], preferred_element_type=jnp.float32) + b2[...],
        0.0)
    mf = u[:, D:] * i[:, D:]
    logit = (jnp.dot(h2, woa[...], preferred_element_type=jnp.float32)
             + jnp.dot(mf, wob[...], preferred_element_type=jnp.float32)
             + bo[...])
    out[...] = jax.nn.sigmoid(logit)


def _tc_dense(gU, gI, parU, parI, W1, b1, W2, b2, Wo, bo):
    w1a, w1b = W1[:D], W1[D:]
    woa, wob = Wo[:32], Wo[32:]
    b1r = b1.reshape(1, -1)
    b2r = b2.reshape(1, -1)
    bor = bo.reshape(1, 1)
    row_spec = pl.BlockSpec((_BS, 128), lambda i: (i, 0))
    par_spec = pl.BlockSpec((_BS, 1), lambda i: (i, 0))
    full = lambda a: pl.BlockSpec(a.shape, lambda i: (0,) * a.ndim)
    out = pl.pallas_call(
        _tc_body,
        grid=(_G,),
        in_specs=[row_spec, row_spec, par_spec, par_spec,
                  full(w1a), full(w1b), full(b1r), full(W2), full(b2r),
                  full(woa), full(wob), full(bor)],
        out_specs=pl.BlockSpec((_BS, 1), lambda i: (i, 0)),
        out_shape=jax.ShapeDtypeStruct((B, 1), jnp.float32),
        compiler_params=pltpu.CompilerParams(
            dimension_semantics=("arbitrary",)),
    )(gU, gI, parU, parI, w1a, w1b, b1r, W2, b2r, woa, wob, bor)
    return out.reshape(B)


def kernel(user_indices, item_indices, embed_user_mlp, embed_item_mlp,
           embed_user_mf, embed_item_mf, W1, b1, W2, b2, Wo, bo):
    vUm = embed_user_mlp.T.reshape(8, 8, NROW)
    vUf = embed_user_mf.T.reshape(8, 8, NROW)
    vIm = embed_item_mlp.T.reshape(8, 8, NROW)
    vIf = embed_item_mf.T.reshape(8, 8, NROW)
    eye = jnp.eye(128, dtype=jnp.float32)
    pU = _tc_pack(vUm, vUf, eye)
    pI = _tc_pack(vIm, vIf, eye)
    gU = _sc_gather(user_indices, pU)
    gI = _sc_gather(item_indices, pI)
    parU = jnp.bitwise_and(user_indices, 1).reshape(B, 1)
    parI = jnp.bitwise_and(item_indices, 1).reshape(B, 1)
    return _tc_dense(gU, gI, parU, parI, W1, b1, W2, b2, Wo, bo)


# speedup vs baseline: 1.0084x; 1.0011x over previous
import functools

import jax
import jax.numpy as jnp
from jax import lax
from jax.experimental import pallas as pl
from jax.experimental.pallas import tpu as pltpu
from jax.experimental.pallas import tpu_sc as plsc

B = 16384
D = 64
NROW = 1000000
RC = 32768
NCHUNK = (NROW + RC - 1) // RC
NPACK = NCHUNK * RC

_info = plsc.get_sparse_core_info()
_NC, _NS = _info.num_cores, _info.num_subcores
_NW = _NC * _NS
_BPW = B // _NW


def _pack_body(xa_ref, xb_ref, o_ref):
    va = xa_ref[...].reshape(D, RC)
    vb = xb_ref[...].reshape(D, RC)
    v = jnp.concatenate([va, vb], axis=0)
    t = v.T
    b16 = t.astype(jnp.bfloat16)
    o_ref[...] = pltpu.bitcast(b16, jnp.float32)


def _tc_pack(view_a, view_b):
    return pl.pallas_call(
        _pack_body,
        grid=(NCHUNK,),
        in_specs=[pl.BlockSpec((8, 8, RC), lambda i: (0, 0, i)),
                  pl.BlockSpec((8, 8, RC), lambda i: (0, 0, i))],
        out_specs=pl.BlockSpec((RC // 2, 128), lambda i: (i, 0)),
        out_shape=jax.ShapeDtypeStruct((NPACK // 2, 128), jnp.float32),
        compiler_params=pltpu.CompilerParams(
            dimension_semantics=("arbitrary",)),
    )(view_a, view_b)


def _sc_gather(idx, tbl):
    mesh = plsc.VectorSubcoreMesh(core_axis_name="c", subcore_axis_name="s")
    f32 = jnp.float32

    @functools.partial(
        pl.kernel,
        mesh=mesh,
        compiler_params=pltpu.CompilerParams(
            use_tc_tiling_on_sc=True, needs_layout_passes=False),
        out_type=jax.ShapeDtypeStruct((B, 128), f32),
        scratch_types=[
            pltpu.VMEM((_BPW,), jnp.int32),
            pltpu.VMEM((4, 128), jnp.int32),
            pltpu.VMEM((_BPW, 128), f32),
            pltpu.SemaphoreType.DMA,
        ],
    )
    def k(idx_hbm, tbl_hbm, out, idx_v, pidx_v, stage, s0):
        wid = lax.axis_index("s") * _NC + lax.axis_index("c")
        base = wid * _BPW
        pltpu.sync_copy(idx_hbm.at[pl.ds(base, _BPW)], idx_v)
        for c in range(_BPW // 16):
            pidx_v[c // 8, pl.ds((c % 8) * 16, 16)] = (
                lax.shift_right_logical(idx_v[pl.ds(c * 16, 16)], 1))
        cps = [pltpu.async_copy(tbl_hbm.at[pidx_v.at[g]],
                                stage.at[pl.ds(g * 128, 128)], s0)
               for g in range(4)]
        for cp in cps:
            cp.wait()
        pltpu.sync_copy(stage, out.at[pl.ds(base, _BPW)])

    return k(idx, tbl)


_BS = 2048
_G = B // _BS


def _unpack_rows(words, parity):
    wi = lax.bitcast_convert_type(words, jnp.int32)
    lo = lax.bitcast_convert_type(lax.shift_left(wi, 16), jnp.float32)
    hi = lax.bitcast_convert_type(
        jnp.bitwise_and(wi, jnp.int32(-65536)), jnp.float32)
    return jnp.where(parity == 1, hi, lo)


def _tc_body(gu, gi, pu, pi, w1a, w1b, b1, w2, b2, woa, wob, bo, out):
    u = _unpack_rows(gu[...], pu[...])
    i = _unpack_rows(gi[...], pi[...])
    h1 = jnp.maximum(
        jnp.dot(u[:, :D], w1a[...], preferred_element_type=jnp.float32)
        + jnp.dot(i[:, :D], w1b[...], preferred_element_type=jnp.float32)
        + b1[...], 0.0)
    h2 = jnp.maximum(
        jnp.dot(h1, w2[...], preferred_element_type=jnp.float32) + b2[...],
        0.0)
    mf = u[:, D:] * i[:, D:]
    logit = (jnp.dot(h2, woa[...], preferred_element_type=jnp.float32)
             + jnp.dot(mf, wob[...], preferred_element_type=jnp.float32)
             + bo[...])
    out[...] = jax.nn.sigmoid(logit)


def _tc_dense(gU, gI, parU, parI, W1, b1, W2, b2, Wo, bo):
    w1a, w1b = W1[:D], W1[D:]
    woa, wob = Wo[:32], Wo[32:]
    b1r = b1.reshape(1, -1)
    b2r = b2.reshape(1, -1)
    bor = bo.reshape(1, 1)
    row_spec = pl.BlockSpec((_BS, 128), lambda i: (i, 0))
    par_spec = pl.BlockSpec((_BS, 1), lambda i: (i, 0))
    full = lambda a: pl.BlockSpec(a.shape, lambda i: (0,) * a.ndim)
    out = pl.pallas_call(
        _tc_body,
        grid=(_G,),
        in_specs=[row_spec, row_spec, par_spec, par_spec,
                  full(w1a), full(w1b), full(b1r), full(W2), full(b2r),
                  full(woa), full(wob), full(bor)],
        out_specs=pl.BlockSpec((_BS, 1), lambda i: (i, 0)),
        out_shape=jax.ShapeDtypeStruct((B, 1), jnp.float32),
        compiler_params=pltpu.CompilerParams(
            dimension_semantics=("arbitrary",)),
    )(gU, gI, parU, parI, w1a, w1b, b1r, W2, b2r, woa, wob, bor)
    return out.reshape(B)


def kernel(user_indices, item_indices, embed_user_mlp, embed_item_mlp,
           embed_user_mf, embed_item_mf, W1, b1, W2, b2, Wo, bo):
    vUm = embed_user_mlp.T.reshape(8, 8, NROW)
    vUf = embed_user_mf.T.reshape(8, 8, NROW)
    vIm = embed_item_mlp.T.reshape(8, 8, NROW)
    vIf = embed_item_mf.T.reshape(8, 8, NROW)
    pU = _tc_pack(vUm, vUf)
    pI = _tc_pack(vIm, vIf)
    gU = _sc_gather(user_indices, pU)
    gI = _sc_gather(item_indices, pI)
    parU = jnp.bitwise_and(user_indices, 1).reshape(B, 1)
    parI = jnp.bitwise_and(item_indices, 1).reshape(B, 1)
    return _tc_dense(gU, gI, parU, parI, W1, b1, W2, b2, Wo, bo)
